# Initial kernel scaffold; baseline (speedup 1.0000x reference)
import jax
import jax.numpy as jnp
from jax.experimental import pallas as pl


def kernel(src, seg, word_table, pos_table, seg_table, gamma, beta):
    raise NotImplementedError("write your pallas kernel here")


# speedup vs baseline: 1.0209x; 1.0209x over previous
import functools

import jax
import jax.numpy as jnp
from jax import lax
from jax.experimental import pallas as pl
from jax.experimental.pallas import tpu as pltpu
from jax.experimental.pallas import tpu_sc as plsc

L = 16
NW = 32
CHUNK = 64
MAGIC = 0x5F3759DF
_PROMISE = jax.lax.GatherScatterMode.PROMISE_IN_BOUNDS

_GDN = jax.lax.GatherDimensionNumbers(
    offset_dims=(), collapsed_slice_dims=(0,), start_index_map=(0,))


def _shuffle(x, idx):
    return lax.gather(x, idx[:, None], _GDN, slice_sizes=(1,),
                      mode=_PROMISE)


def _hsum_splat(x):
    lanes = lax.iota(jnp.int32, L)
    for sh in (8, 4, 2, 1):
        x = x + _shuffle(x, lanes ^ sh)
    return x


def _make_sc_kernel(B, S, V, D, P):
    rows = B * S
    rpw = rows // NW
    nch = rpw // CHUNK
    pchunks = S // CHUNK
    ndg = D // L

    mesh = plsc.VectorSubcoreMesh(core_axis_name="c", subcore_axis_name="s")

    @functools.partial(
        pl.kernel,
        mesh=mesh,
        out_type=jax.ShapeDtypeStruct((rows, D), jnp.float32),
        scratch_types=[
            pltpu.VMEM((P, D), jnp.float32),
            pltpu.VMEM((3, D), jnp.float32),
            pltpu.VMEM((2, D), jnp.float32),
            pltpu.VMEM((D,), jnp.float32),
            pltpu.VMEM((D,), jnp.float32),
            pltpu.VMEM((nch, CHUNK), jnp.int32),
            pltpu.VMEM((CHUNK,), jnp.int32),
            pltpu.VMEM((CHUNK, D), jnp.float32),
            pltpu.SemaphoreType.DMA,
        ],
    )
    def sc_kernel(idx_r, seg_r, wt_r, pt_r, st_r, g_r, b_r, out_r,
                  posv, stv, uv, gv, bv, segv, idxb, wbuf, gsem):
        wid = lax.axis_index("s") * 2 + lax.axis_index("c")

        pltpu.sync_copy(seg_r.at[wid], segv)
        pltpu.sync_copy(pt_r, posv)
        pltpu.sync_copy(st_r, stv)
        pltpu.sync_copy(g_r, gv)
        pltpu.sync_copy(b_r, bv)

        for dg in range(ndg):
            sl = pl.ds(dg * L, L)
            t0 = stv[0, sl]
            t1 = stv[1, sl]
            t2 = stv[2, sl]
            uv[0, sl] = t1 - t0
            uv[1, sl] = t2 - 2.0 * t1 + t0

        def fold_body(p, carry):
            for dg in range(ndg):
                sl = pl.ds(dg * L, L)
                posv[p, sl] = posv[p, sl] + stv[0, sl]
            return carry

        lax.fori_loop(0, P, fold_body, 0)

        def compute_chunk(c):
            pos_base = (c % pchunks) * CHUNK

            def group_body(g, carry):
                sfv = segv[c, pl.ds(g * L, L)].astype(jnp.float32)
                for j in range(L):
                    r = g * L + j
                    sf = sfv[j]
                    c2 = 0.5 * sf * (sf - 1.0)
                    pr = pos_base + r
                    xs = []
                    for dg in range(ndg):
                        sl = pl.ds(dg * L, L)
                        x = (wbuf[r, sl] + posv[pr, sl]
                             + sf * uv[0, sl] + c2 * uv[1, sl])
                        xs.append(x)
                    ss = [xs[k] + xs[k + 4] for k in range(4)]
                    ss = [ss[0] + ss[2], ss[1] + ss[3]]
                    acc = ss[0] + ss[1]
                    qq = [xs[k] * xs[k] + xs[k + 4] * xs[k + 4]
                          for k in range(4)]
                    qq = [qq[0] + qq[2], qq[1] + qq[3]]
                    qcc = qq[0] + qq[1]
                    mean = _hsum_splat(acc) * (1.0 / D)
                    vv = (_hsum_splat(qcc) * (1.0 / D)
                          - mean * mean + 1e-6)
                    yi = lax.bitcast_convert_type(vv, jnp.int32)
                    y0 = lax.bitcast_convert_type(MAGIC - (yi >> 1),
                                                  jnp.float32)
                    xh = vv * 0.5
                    y1 = y0 * (1.5 - xh * y0 * y0)
                    inv = y1 * (1.5 - xh * y1 * y1)
                    for dg in range(ndg):
                        sl = pl.ds(dg * L, L)
                        y = ((xs[dg] - mean) * inv) * gv[sl] + bv[sl]
                        wbuf[r, sl] = y
                return carry

            lax.fori_loop(0, CHUNK // L, group_body, 0)

        def chunk_body(c, carry):
            pltpu.sync_copy(idx_r.at[wid, c], idxb)
            pltpu.make_async_copy(wt_r.at[idxb], wbuf, gsem).start()
            pltpu.make_async_copy(wt_r.at[idxb], wbuf, gsem).wait()
            compute_chunk(c)
            base = wid * rpw + c * CHUNK
            pltpu.sync_copy(wbuf, out_r.at[pl.ds(base, CHUNK)])
            return carry

        lax.fori_loop(0, nch, chunk_body, 0)

    return sc_kernel


def kernel(src, seg, word_table, pos_table, seg_table, gamma, beta):
    B, S = src.shape
    V, D = word_table.shape
    P = pos_table.shape[0]
    rows = B * S
    rpw = rows // NW
    nch = rpw // CHUNK

    idx_r = src.astype(jnp.int32).reshape(NW, nch, CHUNK)
    seg_r = seg.astype(jnp.int32).reshape(NW, nch, CHUNK)

    sc = _make_sc_kernel(B, S, V, D, P)
    out = sc(idx_r, seg_r, word_table, pos_table, seg_table, gamma, beta)
    return out.reshape(B, S, D)
